# baseline (speedup 1.0000x reference)
import functools

import jax
import jax.numpy as jnp
from jax import lax
from jax.experimental import pallas as pl
from jax.experimental.pallas import tpu as pltpu
from jax.experimental.pallas import tpu_sc as plsc

B = 16384
D = 128
N_TAB = 1000
NC = 2
NS = 16
NW = NC * NS
RW = B // NW
CH = 128
NCHUNK = RW // CH
NIB = 2
NOB = 2
L = 16


def _body(embed_h, ts_h, noise_h, a_h, om_h, out_h,
          a_tab, om_tab, ts_v,
          eb0, eb1, nb0, nb1, ob0, ob1,
          es0, es1, ns0, ns1, os0, os1):
    wid = lax.axis_index("s") * NC + lax.axis_index("c")
    base = wid * RW

    ebufs = (eb0, eb1)
    nbufs = (nb0, nb1)
    obufs = (ob0, ob1)
    esems = (es0, es1)
    nsems = (ns0, ns1)
    osems = (os0, os1)

    def start_in(c):
        bsel = c % NIB
        ecp = pltpu.async_copy(embed_h.at[pl.ds(base + c * CH, CH)],
                               ebufs[bsel], esems[bsel])
        ncp = pltpu.async_copy(noise_h.at[pl.ds(base + c * CH, CH)],
                               nbufs[bsel], nsems[bsel])
        return ecp, ncp

    in_cp = [None] * NIB
    out_cp = [None] * NOB
    for c in range(NIB - 1):
        in_cp[c] = start_in(c)

    pltpu.sync_copy(a_h, a_tab)
    pltpu.sync_copy(om_h, om_tab)
    pltpu.sync_copy(ts_h.at[pl.ds(base, RW)], ts_v)

    def compute_chunk(c, ebuf, nbuf, obuf):
        @plsc.parallel_loop(0, CH, unroll=2)
        def rows(r):
            bidx = jnp.full((L,), c * CH + r, dtype=jnp.int32)
            t_b = plsc.load_gather(ts_v, [bidx])
            aj = plsc.load_gather(a_tab, [t_b])
            omj = plsc.load_gather(om_tab, [t_b])
            for k in range(D // L):
                sl = pl.ds(k * L, L)
                obuf[r, sl] = aj * ebuf[r, sl] + omj * nbuf[r, sl]

    for c in range(NCHUNK):
        ib = c % NIB
        ob = c % NOB
        if c + NIB - 1 < NCHUNK:
            in_cp[(c + NIB - 1) % NIB] = start_in(c + NIB - 1)
        ecp, ncp = in_cp[ib]
        ecp.wait()
        ncp.wait()
        if c >= NOB:
            out_cp[ob].wait()
        compute_chunk(c, ebufs[ib], nbufs[ib], obufs[ob])
        out_cp[ob] = pltpu.async_copy(
            obufs[ob], out_h.at[pl.ds(base + c * CH, CH)], osems[ob])

    for c in range(NCHUNK - NOB, NCHUNK):
        out_cp[c % NOB].wait()


@jax.jit
def _diffuse(embed, time_steps, noise, a_tab, om_tab):
    kfn = functools.partial(
        pl.kernel,
        out_type=jax.ShapeDtypeStruct((B, D), jnp.float32),
        mesh=plsc.VectorSubcoreMesh(core_axis_name="c", subcore_axis_name="s"),
        compiler_params=pltpu.CompilerParams(
            needs_layout_passes=False,
            use_tc_tiling_on_sc=False,
        ),
        scratch_types=[
            pltpu.VMEM((N_TAB,), jnp.float32),
            pltpu.VMEM((N_TAB,), jnp.float32),
            pltpu.VMEM((RW,), jnp.int32),
            pltpu.VMEM((CH, D), jnp.float32),
            pltpu.VMEM((CH, D), jnp.float32),
            pltpu.VMEM((CH, D), jnp.float32),
            pltpu.VMEM((CH, D), jnp.float32),
            pltpu.VMEM((CH, D), jnp.float32),
            pltpu.VMEM((CH, D), jnp.float32),
            pltpu.SemaphoreType.DMA,
            pltpu.SemaphoreType.DMA,
            pltpu.SemaphoreType.DMA,
            pltpu.SemaphoreType.DMA,
            pltpu.SemaphoreType.DMA,
            pltpu.SemaphoreType.DMA,
        ],
    )(_body)
    return kfn(embed, time_steps, noise, a_tab, om_tab)


def kernel(embed, time_steps, noise, sqrt_alphas_cumprod,
           sqrt_one_minus_alphas_cumprod):
    ts = time_steps.astype(jnp.int32)
    return _diffuse(embed, ts, noise, sqrt_alphas_cumprod,
                    sqrt_one_minus_alphas_cumprod)


# speedup vs baseline: 1.0079x; 1.0079x over previous
import functools

import jax
import jax.numpy as jnp
from jax import lax
from jax.experimental import pallas as pl
from jax.experimental.pallas import tpu as pltpu
from jax.experimental.pallas import tpu_sc as plsc

B = 16384
D = 128
N_TAB = 1000
NC = 2
NS = 16
NW = NC * NS
RW = B // NW
CH = 128
NCHUNK = RW // CH
NIB = 2
NOB = 2
L = 16


def _body(embed_h, ts_h, noise_h, a_h, om_h, out_h,
          a_tab, om_tab, ts_v,
          eb0, eb1, nb0, nb1, ob0, ob1,
          es0, es1, ns0, ns1, os0, os1):
    wid = lax.axis_index("s") * NC + lax.axis_index("c")
    base = wid * RW

    ebufs = (eb0, eb1)
    nbufs = (nb0, nb1)
    obufs = (ob0, ob1)
    esems = (es0, es1)
    nsems = (ns0, ns1)
    osems = (os0, os1)

    def start_in(c):
        bsel = c % NIB
        ecp = pltpu.async_copy(embed_h.at[pl.ds(base + c * CH, CH)],
                               ebufs[bsel], esems[bsel])
        ncp = pltpu.async_copy(noise_h.at[pl.ds(base + c * CH, CH)],
                               nbufs[bsel], nsems[bsel])
        return ecp, ncp

    in_cp = [None] * NIB
    out_cp = [None] * NOB
    for c in range(NIB - 1):
        in_cp[c] = start_in(c)

    pltpu.sync_copy(a_h, a_tab)
    pltpu.sync_copy(om_h, om_tab)
    pltpu.sync_copy(ts_h.at[pl.ds(base, RW)], ts_v)

    def compute_chunk(c, ebuf, nbuf, obuf):
        @plsc.parallel_loop(0, CH, unroll=1)
        def rows(r):
            bidx = jnp.full((L,), c * CH + r, dtype=jnp.int32)
            t_b = plsc.load_gather(ts_v, [bidx])
            aj = plsc.load_gather(a_tab, [t_b])
            omj = plsc.load_gather(om_tab, [t_b])
            for k in range(D // L):
                sl = pl.ds(k * L, L)
                obuf[r, sl] = aj * ebuf[r, sl] + omj * nbuf[r, sl]

    for c in range(NCHUNK):
        ib = c % NIB
        ob = c % NOB
        if c + NIB - 1 < NCHUNK:
            in_cp[(c + NIB - 1) % NIB] = start_in(c + NIB - 1)
        ecp, ncp = in_cp[ib]
        ecp.wait()
        ncp.wait()
        if c >= NOB:
            out_cp[ob].wait()
        compute_chunk(c, ebufs[ib], nbufs[ib], obufs[ob])
        out_cp[ob] = pltpu.async_copy(
            obufs[ob], out_h.at[pl.ds(base + c * CH, CH)], osems[ob])

    for c in range(NCHUNK - NOB, NCHUNK):
        out_cp[c % NOB].wait()


@jax.jit
def _diffuse(embed, time_steps, noise, a_tab, om_tab):
    kfn = functools.partial(
        pl.kernel,
        out_type=jax.ShapeDtypeStruct((B, D), jnp.float32),
        mesh=plsc.VectorSubcoreMesh(core_axis_name="c", subcore_axis_name="s"),
        compiler_params=pltpu.CompilerParams(needs_layout_passes=False),
        scratch_types=[
            pltpu.VMEM((N_TAB,), jnp.float32),
            pltpu.VMEM((N_TAB,), jnp.float32),
            pltpu.VMEM((RW,), jnp.int32),
            pltpu.VMEM((CH, D), jnp.float32),
            pltpu.VMEM((CH, D), jnp.float32),
            pltpu.VMEM((CH, D), jnp.float32),
            pltpu.VMEM((CH, D), jnp.float32),
            pltpu.VMEM((CH, D), jnp.float32),
            pltpu.VMEM((CH, D), jnp.float32),
            pltpu.SemaphoreType.DMA,
            pltpu.SemaphoreType.DMA,
            pltpu.SemaphoreType.DMA,
            pltpu.SemaphoreType.DMA,
            pltpu.SemaphoreType.DMA,
            pltpu.SemaphoreType.DMA,
        ],
    )(_body)
    return kfn(embed, time_steps, noise, a_tab, om_tab)


def kernel(embed, time_steps, noise, sqrt_alphas_cumprod,
           sqrt_one_minus_alphas_cumprod):
    ts = time_steps.astype(jnp.int32)
    return _diffuse(embed, ts, noise, sqrt_alphas_cumprod,
                    sqrt_one_minus_alphas_cumprod)
